# Initial kernel scaffold; baseline (speedup 1.0000x reference)
import jax
import jax.numpy as jnp
from jax.experimental import pallas as pl


def kernel(output, mask, ind, target):
    raise NotImplementedError("write your pallas kernel here")


# speedup vs baseline: 3.3228x; 3.3228x over previous
import functools

import jax
import jax.numpy as jnp
from jax import lax
from jax.experimental import pallas as pl
from jax.experimental.pallas import tpu as pltpu
from jax.experimental.pallas import tpu_sc as plsc

_L = 16


def _lane_sum(x):
    lanes = jax.lax.broadcasted_iota(jnp.int32, (_L,), 0)
    dnums = lax.GatherDimensionNumbers(
        offset_dims=(), collapsed_slice_dims=(0,), start_index_map=(0,))
    for k in (8, 4, 2, 1):
        perm = (lanes + k) & (_L - 1)
        x = x + lax.gather(x, perm[:, None], dnums, slice_sizes=(1,),
                           mode=lax.GatherScatterMode.PROMISE_IN_BOUNDS)
    return x


def _make_sc_kernel(B, C, N, KPAD, CH, NW):
    WPB = KPAD // CH

    def body(flat_h, ind_h, mask_h, targ_h, out_h,
             idx0, idx1, val0, val1, ind_v, mask_v, t0, t1,
             part_v, shared, accbuf, out_v, sem):
        w = lax.axis_index("s")
        b = w // WPB
        koff = (w % WPB) * CH
        base = b * KPAD + koff
        pltpu.sync_copy(ind_h.at[pl.ds(base, CH)], ind_v)
        pltpu.sync_copy(mask_h.at[pl.ds(base, CH)], mask_v)
        pltpu.sync_copy(targ_h.at[pl.ds((b * C + 0) * KPAD + koff, CH)], t0)
        pltpu.sync_copy(targ_h.at[pl.ds((b * C + 1) * KPAD + koff, CH)], t1)

        off0 = (b * C + 0) * N
        off1 = (b * C + 1) * N
        for j in range(CH // _L):
            n = ind_v[pl.ds(j * _L, _L)]
            idx0[pl.ds(j * _L, _L)] = n + off0
            idx1[pl.ds(j * _L, _L)] = n + off1

        cp0 = pltpu.async_copy(flat_h.at[idx0], val0, sem)
        cp1 = pltpu.async_copy(flat_h.at[idx1], val1, sem)
        cp0.wait()
        cp1.wait()

        accl = jnp.zeros((_L,), jnp.float32)
        accm = jnp.zeros((_L,), jnp.float32)
        for j in range(CH // _L):
            sl = pl.ds(j * _L, _L)
            mk = mask_v[sl]
            accm = accm + mk + mk
            accl = accl + jnp.abs(val0[sl] - t0[sl]) * mk
            accl = accl + jnp.abs(val1[sl] - t1[sl]) * mk

        part_v[pl.ds(0, _L)] = accl
        part_v[pl.ds(_L, _L)] = accm
        pltpu.sync_copy(part_v, shared.at[pl.ds(w * 2 * _L, 2 * _L)])
        plsc.subcore_barrier()

        @pl.when(w == 0)
        def _():
            pltpu.sync_copy(shared, accbuf)
            al = jnp.zeros((_L,), jnp.float32)
            am = jnp.zeros((_L,), jnp.float32)
            for i in range(NW):
                al = al + accbuf[pl.ds(i * 2 * _L, _L)]
                am = am + accbuf[pl.ds(i * 2 * _L + _L, _L)]
            al = _lane_sum(al)
            am = _lane_sum(am)
            out_v[...] = al / (am + 1e-4)
            pltpu.sync_copy(out_v, out_h)

    mesh = plsc.VectorSubcoreMesh(
        core_axis_name="c", subcore_axis_name="s", num_cores=1)
    return pl.kernel(
        body,
        out_type=jax.ShapeDtypeStruct((_L,), jnp.float32),
        mesh=mesh,
        scratch_types=[
            pltpu.VMEM((CH,), jnp.int32),
            pltpu.VMEM((CH,), jnp.int32),
            pltpu.VMEM((CH,), jnp.float32),
            pltpu.VMEM((CH,), jnp.float32),
            pltpu.VMEM((CH,), jnp.int32),
            pltpu.VMEM((CH,), jnp.float32),
            pltpu.VMEM((CH,), jnp.float32),
            pltpu.VMEM((CH,), jnp.float32),
            pltpu.VMEM((2 * _L,), jnp.float32),
            pltpu.VMEM_SHARED((NW * 2 * _L,), jnp.float32),
            pltpu.VMEM((NW * 2 * _L,), jnp.float32),
            pltpu.VMEM((_L,), jnp.float32),
            pltpu.SemaphoreType.DMA,
        ],
    )


@functools.partial(jax.jit, static_argnums=())
def kernel(output, mask, ind, target):
    B, C, D, H, W = output.shape
    K = ind.shape[1]
    N = D * H * W
    NW = 16
    WPB = NW // B
    CH = -(-K // WPB)
    CH = -(CH // -_L) * _L
    if CH % 8:
        CH += 8 - CH % 8
    KPAD = CH * WPB

    flat = output.reshape(B * C * N)
    ind32 = jnp.pad(ind.astype(jnp.int32), ((0, 0), (0, KPAD - K)))
    maskf = jnp.pad(mask, ((0, 0), (0, KPAD - K))).astype(jnp.float32)
    targ = jnp.pad(jnp.transpose(target, (0, 2, 1)),
                   ((0, 0), (0, 0), (0, KPAD - K)))

    fn = _make_sc_kernel(B, C, N, KPAD, CH, NW)
    res = fn(flat, ind32.reshape(-1), maskf.reshape(-1), targ.reshape(-1))
    return res[0]


# speedup vs baseline: 3.7189x; 1.1192x over previous
import functools

import jax
import jax.numpy as jnp
from jax import lax
from jax.experimental import pallas as pl
from jax.experimental.pallas import tpu as pltpu
from jax.experimental.pallas import tpu_sc as plsc

_L = 16


def _lane_sum(x):
    lanes = lax.broadcasted_iota(jnp.int32, (_L,), 0)
    dnums = lax.GatherDimensionNumbers(
        offset_dims=(), collapsed_slice_dims=(0,), start_index_map=(0,))
    for k in (8, 4, 2, 1):
        perm = (lanes + k) & (_L - 1)
        x = x + lax.gather(x, perm[:, None], dnums, slice_sizes=(1,),
                           mode=lax.GatherScatterMode.PROMISE_IN_BOUNDS)
    return x


def _make_sc_kernel(B, C, N, KPAD, CH, NW):
    WPB = KPAD // CH
    NV = CH // _L

    def body(flat_h, aux_h, out_h,
             aux_v, idx0, idx1, val0, val1, part_v, shared, accbuf, sem):
        w = lax.axis_index("s")
        b = w // WPB
        pltpu.sync_copy(aux_h.at[pl.ds(w * 4 * CH, 4 * CH)], aux_v)

        off0 = (b * C + 0) * N
        off1 = (b * C + 1) * N

        def build(j, _):
            n = aux_v[pl.ds(j * _L, _L)].astype(jnp.int32)
            idx0[pl.ds(j * _L, _L)] = n + off0
            idx1[pl.ds(j * _L, _L)] = n + off1
            return 0

        lax.fori_loop(0, NV, build, 0, unroll=False)

        cp0 = pltpu.async_copy(flat_h.at[idx0], val0, sem)
        cp1 = pltpu.async_copy(flat_h.at[idx1], val1, sem)
        cp0.wait()
        cp1.wait()

        def accum(j, carry):
            accl, accm = carry
            o = j * _L
            mk = aux_v[pl.ds(CH + o, _L)]
            t0 = aux_v[pl.ds(2 * CH + o, _L)]
            t1 = aux_v[pl.ds(3 * CH + o, _L)]
            accl = accl + (jnp.abs(val0[pl.ds(o, _L)] - t0)
                           + jnp.abs(val1[pl.ds(o, _L)] - t1)) * mk
            accm = accm + mk + mk
            return accl, accm

        zero = jnp.zeros((_L,), jnp.float32)
        accl, accm = lax.fori_loop(0, NV, accum, (zero, zero), unroll=False)

        part_v[pl.ds(0, _L)] = accl
        part_v[pl.ds(_L, _L)] = accm
        pltpu.sync_copy(part_v, shared.at[pl.ds(w * 2 * _L, 2 * _L)])
        plsc.subcore_barrier()

        @pl.when(w == 0)
        def _():
            pltpu.sync_copy(shared, accbuf)

            def comb(i, carry):
                al, am = carry
                return (al + accbuf[pl.ds(i * 2 * _L, _L)],
                        am + accbuf[pl.ds(i * 2 * _L + _L, _L)])

            al, am = lax.fori_loop(0, NW, comb, (zero, zero), unroll=False)
            al = _lane_sum(al)
            am = _lane_sum(am)
            part_v[pl.ds(0, _L)] = al / (am + 1e-4)
            pltpu.sync_copy(part_v.at[pl.ds(0, _L)], out_h)

    mesh = plsc.VectorSubcoreMesh(
        core_axis_name="c", subcore_axis_name="s", num_cores=1)
    return pl.kernel(
        body,
        out_type=jax.ShapeDtypeStruct((_L,), jnp.float32),
        mesh=mesh,
        scratch_types=[
            pltpu.VMEM((4 * CH,), jnp.float32),
            pltpu.VMEM((CH,), jnp.int32),
            pltpu.VMEM((CH,), jnp.int32),
            pltpu.VMEM((CH,), jnp.float32),
            pltpu.VMEM((CH,), jnp.float32),
            pltpu.VMEM((2 * _L,), jnp.float32),
            pltpu.VMEM_SHARED((NW * 2 * _L,), jnp.float32),
            pltpu.VMEM((NW * 2 * _L,), jnp.float32),
            pltpu.SemaphoreType.DMA,
        ],
    )


@jax.jit
def kernel(output, mask, ind, target):
    B, C, D, H, W = output.shape
    K = ind.shape[1]
    N = D * H * W
    NW = 16
    WPB = NW // B
    CH = -(-K // WPB)
    CH = -(CH // -_L) * _L
    if CH % 8:
        CH += 8 - CH % 8
    KPAD = CH * WPB

    flat = output.reshape(B * C * N)
    pad = ((0, 0), (0, KPAD - K))
    indf = jnp.pad(ind.astype(jnp.float32), pad).reshape(NW, 1, CH)
    mk = jnp.pad(mask, pad).astype(jnp.float32).reshape(NW, 1, CH)
    tg = jnp.pad(jnp.transpose(target, (0, 2, 1)),
                 ((0, 0), (0, 0), (0, KPAD - K)))
    tg = tg.reshape(B, C, WPB, CH).transpose(0, 2, 1, 3).reshape(NW, C, CH)
    aux = jnp.concatenate([indf, mk, tg], axis=1).reshape(-1)

    fn = _make_sc_kernel(B, C, N, KPAD, CH, NW)
    res = fn(flat, aux)
    return res[0]
